# Initial kernel scaffold; baseline (speedup 1.0000x reference)
import jax
import jax.numpy as jnp
from jax.experimental import pallas as pl


def kernel(node_s, node_v, edge_s, edge_v, frames, W_vdown, W_vframes, W_sout, b_sout, W_vup, W_gate, b_gate, edge_index):
    raise NotImplementedError("write your pallas kernel here")


# speedup vs baseline: 8.4266x; 8.4266x over previous
import jax
import jax.numpy as jnp
from jax import lax
from jax.experimental import pallas as pl
from jax.experimental.pallas import tpu as pltpu
from jax.experimental.pallas import tpu_sc as plsc

_N = 10000
_NP = 10240
_E = 160000
_DS = 128
_DV = 16
_DES = 32
_HID = 36

_NC = 2
_NS = 16
_NW = _NC * _NS
_EPW = _E // _NW
_CH = 40
_NCHUNK = _EPW // _CH

_EPT4 = _E // _NS
_NCH4 = _EPT4 // _CH
_RPT = _NP // _NS


def _prep_body(ns_ref, whs_ref, pr_ref, pc_ref):
    p = jnp.dot(ns_ref[...], whs_ref[...], preferred_element_type=jnp.float32)
    pr_ref[...] = p[:, :_DS]
    pc_ref[...] = p[:, _DS:]


def _precompute_tables(node_s, whs):
    bn = 2000
    return pl.pallas_call(
        _prep_body,
        grid=(_N // bn,),
        in_specs=[
            pl.BlockSpec((bn, _DS), lambda i: (i, 0)),
            pl.BlockSpec((_DS, 2 * _DS), lambda i: (0, 0)),
        ],
        out_specs=[
            pl.BlockSpec((bn, _DS), lambda i: (i, 0)),
            pl.BlockSpec((bn, _DS), lambda i: (i, 0)),
        ],
        out_shape=[
            jax.ShapeDtypeStruct((_N, _DS), jnp.float32),
            jax.ShapeDtypeStruct((_N, _DS), jnp.float32),
        ],
    )(node_s, whs)


def _gather_body(row_hbm, col_hbm, pr_hbm, pc_hbm, vt_hbm,
                 gpr_hbm, gpc_hbm, gvr_hbm, gvc_hbm,
                 idx_r, idx_c, buf_pr, buf_pc, buf_vr, buf_vc, sem):
    c = lax.axis_index("c")
    s = lax.axis_index("s")
    wid = s * _NC + c

    def step(i, carry):
        base = wid * _EPW + i * _CH
        pltpu.sync_copy(row_hbm.at[pl.ds(base, _CH)], idx_r)
        pltpu.sync_copy(col_hbm.at[pl.ds(base, _CH)], idx_c)
        cp1 = pltpu.async_copy(pr_hbm.at[idx_r], buf_pr, sem)
        cp2 = pltpu.async_copy(pc_hbm.at[idx_c], buf_pc, sem)
        cp3 = pltpu.async_copy(vt_hbm.at[idx_r], buf_vr, sem)
        cp4 = pltpu.async_copy(vt_hbm.at[idx_c], buf_vc, sem)
        cp1.wait()
        cp2.wait()
        cp3.wait()
        cp4.wait()
        pltpu.sync_copy(buf_pr, gpr_hbm.at[pl.ds(base, _CH)])
        pltpu.sync_copy(buf_pc, gpc_hbm.at[pl.ds(base, _CH)])
        pltpu.sync_copy(buf_vr, gvr_hbm.at[pl.ds(base, _CH)])
        pltpu.sync_copy(buf_vc, gvc_hbm.at[pl.ds(base, _CH)])
        return carry

    lax.fori_loop(0, _NCHUNK, step, 0)


def _gather_edges(row, col, p_r, p_c, vt):
    mesh = plsc.VectorSubcoreMesh(core_axis_name="c", subcore_axis_name="s",
                                  num_cores=_NC, num_subcores=_NS)
    call = pl.kernel(
        _gather_body,
        out_type=(
            jax.ShapeDtypeStruct((_E, _DS), jnp.float32),
            jax.ShapeDtypeStruct((_E, _DS), jnp.float32),
            jax.ShapeDtypeStruct((_E, _DS), jnp.float32),
            jax.ShapeDtypeStruct((_E, _DS), jnp.float32),
        ),
        mesh=mesh,
        scratch_types=[
            pltpu.VMEM((_CH,), jnp.int32),
            pltpu.VMEM((_CH,), jnp.int32),
            pltpu.VMEM((_CH, _DS), jnp.float32),
            pltpu.VMEM((_CH, _DS), jnp.float32),
            pltpu.VMEM((_CH, _DS), jnp.float32),
            pltpu.VMEM((_CH, _DS), jnp.float32),
            pltpu.SemaphoreType.DMA,
        ],
    )
    return call(row, col, p_r, p_c, vt)


_B3 = 1000


def _edge_body(gpr, gpc, gvr, gvc, es, evt, fr,
               wbr, wbc, wbe, wes, wvn, wscal, bs, wg, bg, out):
    m = []
    nrm2 = None
    for a in range(3):
        ma = jnp.dot(gvr[:, 16 * a:16 * a + 16], wbr[...],
                     preferred_element_type=jnp.float32)
        ma = ma + jnp.dot(gvc[:, 16 * a:16 * a + 16], wbc[...],
                          preferred_element_type=jnp.float32)
        ma = ma + jnp.dot(evt[:, 4 * a:4 * a + 4], wbe[...],
                          preferred_element_type=jnp.float32)
        m.append(ma)
        sq = ma[:, :_HID] * ma[:, :_HID]
        nrm2 = sq if nrm2 is None else nrm2 + sq
    vnorm = jnp.sqrt(nrm2 + 1e-8)
    slin = gpr[...] + gpc[...] + bs[...]
    slin = slin + jnp.dot(es[...], wes[...], preferred_element_type=jnp.float32)
    slin = slin + jnp.dot(vnorm, wvn[...], preferred_element_type=jnp.float32)
    for cc in range(3):
        for aa in range(3):
            t = m[0][:, _HID + cc:_HID + cc + 1] * fr[:, 3 * aa:3 * aa + 1]
            t = t + m[1][:, _HID + cc:_HID + cc + 1] * fr[:, 3 * aa + 1:3 * aa + 2]
            t = t + m[2][:, _HID + cc:_HID + cc + 1] * fr[:, 3 * aa + 2:3 * aa + 3]
            slin = slin + t * wscal[3 * cc + aa:3 * cc + aa + 1, :]
    sact = slin * jax.nn.sigmoid(slin)
    gate = jax.nn.sigmoid(
        jnp.dot(sact, wg[...], preferred_element_type=jnp.float32) + bg[...])
    out[0] = sact
    vparts = [m[a][:, 39:39 + _DV] * gate for a in range(3)]
    vparts.append(jnp.zeros((sact.shape[0], _DS - 3 * _DV), jnp.float32))
    out[1] = jnp.concatenate(vparts, axis=1)


def _edge_compute(gpr, gpc, gvr, gvc, edge_s, evt, fr,
                  wbr, wbc, wbe, wes, wvn, wscal, bs, wg, bg):
    full = lambda shape: pl.BlockSpec(shape, lambda i: tuple(0 for _ in shape))
    return pl.pallas_call(
        _edge_body,
        grid=(_E // _B3,),
        in_specs=[
            pl.BlockSpec((_B3, _DS), lambda i: (i, 0)),
            pl.BlockSpec((_B3, _DS), lambda i: (i, 0)),
            pl.BlockSpec((_B3, _DS), lambda i: (i, 0)),
            pl.BlockSpec((_B3, _DS), lambda i: (i, 0)),
            pl.BlockSpec((_B3, _DES), lambda i: (i, 0)),
            pl.BlockSpec((_B3, 12), lambda i: (i, 0)),
            pl.BlockSpec((_B3, 9), lambda i: (i, 0)),
            full((16, 55)),
            full((16, 55)),
            full((4, 55)),
            full((_DES, _DS)),
            full((_HID, _DS)),
            full((9, _DS)),
            full((1, _DS)),
            full((_DS, _DV)),
            full((1, _DV)),
        ],
        out_specs=pl.BlockSpec((2, _B3, _DS), lambda i: (0, i, 0)),
        out_shape=jax.ShapeDtypeStruct((2, _E, _DS), jnp.float32),
    )(gpr, gpc, gvr, gvc, edge_s, evt, fr,
      wbr, wbc, wbe, wes, wvn, wscal, bs, wg, bg)


def _scatter_body(fsv_hbm, row_hbm, part_hbm, acc_sh, zbuf, buf, idx_v, sem):
    c = lax.axis_index("c")
    s = lax.axis_index("s")

    def zstore(i, carry):
        for j in range(_DS // 16):
            zbuf[i, pl.ds(j * 16, 16)] = jnp.zeros((16,), jnp.float32)
        return carry

    lax.fori_loop(0, _CH, zstore, 0)

    def zcopy(k, carry):
        pltpu.sync_copy(zbuf, acc_sh.at[pl.ds(s * _RPT + k * _CH, _CH)])
        return carry

    lax.fori_loop(0, _RPT // _CH, zcopy, 0)
    plsc.subcore_barrier()

    def step(i, carry):
        base = s * _EPT4 + i * _CH
        pltpu.sync_copy(row_hbm.at[pl.ds(base, _CH)], idx_v)
        pltpu.sync_copy(fsv_hbm.at[c, pl.ds(base, _CH)], buf)
        pltpu.sync_copy(buf, acc_sh.at[idx_v], add=True)
        return carry

    lax.fori_loop(0, _NCH4, step, 0)
    plsc.subcore_barrier()
    pltpu.sync_copy(acc_sh.at[pl.ds(s * _RPT, _RPT)],
                    part_hbm.at[c, pl.ds(s * _RPT, _RPT)])


def _scatter_edges(fsv, row):
    mesh = plsc.VectorSubcoreMesh(core_axis_name="c", subcore_axis_name="s",
                                  num_cores=_NC, num_subcores=_NS)
    call = pl.kernel(
        _scatter_body,
        out_type=jax.ShapeDtypeStruct((_NC, _NP, _DS), jnp.float32),
        mesh=mesh,
        scratch_types=[
            pltpu.VMEM_SHARED((_NP, _DS), jnp.float32),
            pltpu.VMEM((_CH, _DS), jnp.float32),
            pltpu.VMEM((_CH, _DS), jnp.float32),
            pltpu.VMEM((_CH,), jnp.int32),
            pltpu.SemaphoreType.DMA,
        ],
    )
    return call(fsv, row)


def _final_body(ps_ref, pv_ref, outs_ref, outv_ref):
    outs_ref[...] = ps_ref[0]
    v = pv_ref[0][:, :3 * _DV].reshape(ps_ref.shape[1], 3, _DV)
    outv_ref[...] = jnp.swapaxes(v, 1, 2)


def _combine(part):
    bn = 400
    return pl.pallas_call(
        _final_body,
        grid=(_N // bn,),
        in_specs=[
            pl.BlockSpec((1, bn, _DS), lambda i: (0, i, 0)),
            pl.BlockSpec((1, bn, _DS), lambda i: (1, i, 0)),
        ],
        out_specs=[
            pl.BlockSpec((bn, _DS), lambda i: (i, 0)),
            pl.BlockSpec((bn, _DV, 3), lambda i: (i, 0, 0)),
        ],
        out_shape=[
            jax.ShapeDtypeStruct((_N, _DS), jnp.float32),
            jax.ShapeDtypeStruct((_N, _DV, 3), jnp.float32),
        ],
    )(part, part)


@jax.jit
def _run(node_s, node_v, edge_s, edge_v, frames, W_vdown, W_vframes,
         W_sout, b_sout, W_vup, W_gate, b_gate, edge_index):
    row = edge_index[0]
    col = edge_index[1]

    whs = jnp.concatenate([W_sout[:_DS], W_sout[_DS:2 * _DS]], axis=1)
    w_upf = W_vdown @ W_vup
    w_big = jnp.concatenate([W_vdown, W_vframes, w_upf], axis=1)
    wbr = w_big[:_DV]
    wbc = w_big[_DV:2 * _DV]
    wbe = w_big[2 * _DV:]
    wes = W_sout[2 * _DS:2 * _DS + _DES]
    wvn = W_sout[2 * _DS + _DES:2 * _DS + _DES + _HID]
    wscal = W_sout[2 * _DS + _DES + _HID:]
    bs = b_sout.reshape(1, _DS)
    bg = b_gate.reshape(1, _DV)

    vt = jnp.swapaxes(node_v, 1, 2).reshape(_N, 3 * _DV)
    vt = jnp.pad(vt, ((0, 0), (0, _DS - 3 * _DV)))
    evt = jnp.swapaxes(edge_v, 1, 2).reshape(_E, 12)
    fr = frames.reshape(_E, 9)

    p_r, p_c = _precompute_tables(node_s, whs)
    gpr, gpc, gvr, gvc = _gather_edges(row, col, p_r, p_c, vt)
    fsv = _edge_compute(gpr, gpc, gvr, gvc, edge_s, evt, fr,
                        wbr, wbc, wbe, wes, wvn, wscal, bs, W_gate, bg)
    part = _scatter_edges(fsv, row)
    agg_s, agg_v = _combine(part)
    return agg_s, agg_v


def kernel(node_s, node_v, edge_s, edge_v, frames, W_vdown, W_vframes,
           W_sout, b_sout, W_vup, W_gate, b_gate, edge_index):
    return _run(node_s, node_v, edge_s, edge_v, frames, W_vdown, W_vframes,
                W_sout, b_sout, W_vup, W_gate, b_gate, edge_index)
